# baseline (speedup 1.0000x reference)
import functools

import jax
import jax.numpy as jnp
from jax.experimental import pallas as pl


_B, _C2, _N = 16, 128, 1024
_C = _C2 // 2
_Q, _K, _D = 4, 1024, 64

@functools.lru_cache(maxsize=1)
def _noise_const():
    import numpy as np
    with jax.ensure_compile_time_eval():
        return np.asarray(
            jax.random.normal(jax.random.key(42), (_B, _C, _N), dtype=jnp.float32))


def _rvq_kernel(x_ref, noise_ref, cb_ref, o_ref):
    xb = x_ref[0]
    mean = xb[:_C, :]
    scale = xb[_C:, :]
    sp = jnp.maximum(scale, 0.0) + jnp.log1p(jnp.exp(-jnp.abs(scale)))
    stdev = sp + 0.0001
    r = noise_ref[0] * stdev + mean

    nb = r.shape[1]
    row_iota = jax.lax.broadcasted_iota(jnp.int32, (_K, nb), 0).astype(jnp.float32)

    acc = jnp.zeros((_C, nb), dtype=jnp.float32)
    for q in range(_Q):
        cb = cb_ref[q]
        c2 = jnp.sum(cb * cb, axis=1)[:, None]
        r2 = jnp.sum(r * r, axis=0, keepdims=True)
        cbm2 = -2.0 * cb
        c_hi16 = cb.astype(jnp.bfloat16)
        rem1 = cb - c_hi16.astype(jnp.float32)
        c_mid16 = rem1.astype(jnp.bfloat16)
        c_lo16 = (rem1 - c_mid16.astype(jnp.float32)).astype(jnp.bfloat16)
        c_stack = jnp.concatenate([c_hi16, c_mid16, c_lo16], axis=1)
        scores = jax.lax.dot_general(
            cbm2, r, (((1,), (0,)), ((), ())),
            preferred_element_type=jnp.float32)
        d = (r2 + scores) + c2
        m = jnp.min(d, axis=0, keepdims=True)
        w = jnp.where(d == m, row_iota, float(_K))
        idx = jnp.min(w, axis=0, keepdims=True)
        onehot = (row_iota == idx).astype(jnp.float32).astype(jnp.bfloat16)
        parts = jax.lax.dot_general(
            c_stack, onehot, (((0,), (0,)), ((), ())),
            preferred_element_type=jnp.float32)
        quant = (parts[0:_C] + parts[_C:2 * _C]) + parts[2 * _C:3 * _C]
        acc = acc + quant
        r = r - quant

    o_ref[0] = acc


@functools.partial(jax.jit, static_argnames=())
def kernel(x, codebooks):
    noise = jnp.asarray(_noise_const())
    nb = 1024
    grid = (_B, _N // nb)
    return pl.pallas_call(
        _rvq_kernel,
        grid=grid,
        in_specs=[
            pl.BlockSpec((1, _C2, nb), lambda b, j: (b, 0, j)),
            pl.BlockSpec((1, _C, nb), lambda b, j: (b, 0, j)),
            pl.BlockSpec((_Q, _K, _D), lambda b, j: (0, 0, 0)),
        ],
        out_specs=pl.BlockSpec((1, _C, nb), lambda b, j: (b, 0, j)),
        out_shape=jax.ShapeDtypeStruct((_B, _C, _N), jnp.float32),
    )(x, noise, codebooks)


# speedup vs baseline: 1.0917x; 1.0917x over previous
import functools

import jax
import jax.numpy as jnp
from jax.experimental import pallas as pl


_B, _C2, _N = 16, 128, 1024
_C = _C2 // 2
_Q, _K, _D = 4, 1024, 64

@functools.lru_cache(maxsize=1)
def _noise_const():
    import numpy as np
    with jax.ensure_compile_time_eval():
        return np.asarray(
            jax.random.normal(jax.random.key(42), (_B, _C, _N), dtype=jnp.float32))


def _rvq_kernel(x_ref, noise_ref, cb_ref, o_ref):
    nbat = x_ref.shape[0]
    rs = []
    for b in range(nbat):
        xb = x_ref[b]
        mean = xb[:_C, :]
        scale = xb[_C:, :]
        sp = jnp.maximum(scale, 0.0) + jnp.log1p(jnp.exp(-jnp.abs(scale)))
        stdev = sp + 0.0001
        rs.append(noise_ref[b] * stdev + mean)
    r = jnp.concatenate(rs, axis=1) if nbat > 1 else rs[0]

    nb = r.shape[1]
    row_iota = jax.lax.broadcasted_iota(jnp.int32, (_K, nb), 0).astype(jnp.float32)

    acc = jnp.zeros((_C, nb), dtype=jnp.float32)
    for q in range(_Q):
        cb = cb_ref[q]
        c2 = jnp.sum(cb * cb, axis=1)[:, None]
        r2 = jnp.sum(r * r, axis=0, keepdims=True)
        cbm2 = -2.0 * cb
        c_hi16 = cb.astype(jnp.bfloat16)
        rem1 = cb - c_hi16.astype(jnp.float32)
        c_mid16 = rem1.astype(jnp.bfloat16)
        c_lo16 = (rem1 - c_mid16.astype(jnp.float32)).astype(jnp.bfloat16)
        c_stack = jnp.concatenate([c_hi16, c_mid16, c_lo16], axis=1)
        scores = jax.lax.dot_general(
            cbm2, r, (((1,), (0,)), ((), ())),
            preferred_element_type=jnp.float32)
        d = (r2 + scores) + c2
        m = jnp.min(d, axis=0, keepdims=True)
        w = jnp.where(d == m, row_iota, float(_K))
        idx = jnp.min(w, axis=0, keepdims=True)
        onehot = (row_iota == idx).astype(jnp.float32).astype(jnp.bfloat16)
        parts = jax.lax.dot_general(
            c_stack, onehot, (((0,), (0,)), ((), ())),
            preferred_element_type=jnp.float32)
        quant = (parts[0:_C] + parts[_C:2 * _C]) + parts[2 * _C:3 * _C]
        acc = acc + quant
        r = r - quant

    for b in range(nbat):
        o_ref[b] = acc[:, b * _N:(b + 1) * _N]


@functools.partial(jax.jit, static_argnames=())
def kernel(x, codebooks):
    noise = jnp.asarray(_noise_const())
    nbat = 2
    grid = (_B // nbat,)
    return pl.pallas_call(
        _rvq_kernel,
        grid=grid,
        in_specs=[
            pl.BlockSpec((nbat, _C2, _N), lambda b: (b, 0, 0)),
            pl.BlockSpec((nbat, _C, _N), lambda b: (b, 0, 0)),
            pl.BlockSpec((_Q, _K, _D), lambda b: (0, 0, 0)),
        ],
        out_specs=pl.BlockSpec((nbat, _C, _N), lambda b: (b, 0, 0)),
        out_shape=jax.ShapeDtypeStruct((_B, _C, _N), jnp.float32),
    )(x, noise, codebooks)


# speedup vs baseline: 1.1153x; 1.0217x over previous
import functools

import jax
import jax.numpy as jnp
from jax.experimental import pallas as pl


_B, _C2, _N = 16, 128, 1024
_C = _C2 // 2
_Q, _K, _D = 4, 1024, 64

@functools.lru_cache(maxsize=1)
def _noise_const():
    import numpy as np
    with jax.ensure_compile_time_eval():
        return np.asarray(
            jax.random.normal(jax.random.key(42), (_B, _C, _N), dtype=jnp.float32))


def _rvq_kernel(x_ref, noise_ref, cb_ref, o_ref):
    nbat = x_ref.shape[0]
    rs = []
    for b in range(nbat):
        xb = x_ref[b]
        mean = xb[:_C, :]
        scale = xb[_C:, :]
        sp = jnp.maximum(scale, 0.0) + jnp.log1p(jnp.exp(-jnp.abs(scale)))
        stdev = sp + 0.0001
        rs.append(noise_ref[b] * stdev + mean)
    r = jnp.concatenate(rs, axis=1) if nbat > 1 else rs[0]

    nb = r.shape[1]
    row_iota = jax.lax.broadcasted_iota(jnp.int32, (_K, nb), 0).astype(jnp.float32)

    acc = jnp.zeros((_C, nb), dtype=jnp.float32)
    for q in range(_Q):
        cb = cb_ref[q]
        c2 = jnp.sum(cb * cb, axis=1)[:, None]
        r2 = jnp.sum(r * r, axis=0, keepdims=True)
        cbm2 = -2.0 * cb
        c_hi16 = cb.astype(jnp.bfloat16)
        rem1 = cb - c_hi16.astype(jnp.float32)
        c_mid16 = rem1.astype(jnp.bfloat16)
        c_lo16 = (rem1 - c_mid16.astype(jnp.float32)).astype(jnp.bfloat16)
        c_stack = jnp.concatenate([c_hi16, c_mid16, c_lo16], axis=1)
        scores = jax.lax.dot_general(
            cbm2, r, (((1,), (0,)), ((), ())),
            preferred_element_type=jnp.float32)
        d = (r2 + scores) + c2
        m = jnp.min(d, axis=0, keepdims=True)
        w = jnp.where(d == m, row_iota, float(_K))
        idx = jnp.min(w, axis=0, keepdims=True)
        onehot = (row_iota == idx).astype(jnp.float32).astype(jnp.bfloat16)
        parts = jax.lax.dot_general(
            c_stack, onehot, (((0,), (0,)), ((), ())),
            preferred_element_type=jnp.float32)
        quant = (parts[0:_C] + parts[_C:2 * _C]) + parts[2 * _C:3 * _C]
        acc = acc + quant
        r = r - quant

    for b in range(nbat):
        o_ref[b] = acc[:, b * _N:(b + 1) * _N]


@functools.partial(jax.jit, static_argnames=())
def kernel(x, codebooks):
    noise = jnp.asarray(_noise_const())
    nbat = 4
    grid = (_B // nbat,)
    return pl.pallas_call(
        _rvq_kernel,
        grid=grid,
        in_specs=[
            pl.BlockSpec((nbat, _C2, _N), lambda b: (b, 0, 0)),
            pl.BlockSpec((nbat, _C, _N), lambda b: (b, 0, 0)),
            pl.BlockSpec((_Q, _K, _D), lambda b: (0, 0, 0)),
        ],
        out_specs=pl.BlockSpec((nbat, _C, _N), lambda b: (b, 0, 0)),
        out_shape=jax.ShapeDtypeStruct((_B, _C, _N), jnp.float32),
    )(x, noise, codebooks)
